# Initial kernel scaffold; baseline (speedup 1.0000x reference)
import jax
import jax.numpy as jnp
from jax.experimental import pallas as pl


def kernel(right_embed, edge_index, adj_vals, perm, gcnW1, highwayWr, highwaybr):
    raise NotImplementedError("write your pallas kernel here")


# speedup vs baseline: 3.6160x; 3.6160x over previous
import functools

import jax
import jax.numpy as jnp
from jax import lax
from jax.experimental import pallas as pl
from jax.experimental.pallas import tpu as pltpu
from jax.experimental.pallas import tpu_sc as plsc

N = 10000
E = 320000
D = 128

NC, NS, L = 2, 16, 16
NW = NC * NS
CHUNK = 128
CPT = 79
EPT = CPT * CHUNK
EPAD = NW * EPT
PCH = 64
PCPT = 5
PPT = PCH * PCPT
NPAD = NW * PPT
NACC = 10240
RPS = NACC // NS
ZR = 128


def _mm2_body(x_ref, w1_ref, w2_ref, o1_ref, o2_ref):
    x = x_ref[...]
    o1_ref[...] = jnp.dot(x, w1_ref[...], preferred_element_type=jnp.float32)
    o2_ref[...] = jnp.dot(x, w2_ref[...], preferred_element_type=jnp.float32)


def _mm2(x, w1, w2):
    BM = 1000
    return pl.pallas_call(
        _mm2_body,
        grid=(N // BM,),
        in_specs=[pl.BlockSpec((BM, D), lambda i: (i, 0)),
                  pl.BlockSpec((D, D), lambda i: (0, 0)),
                  pl.BlockSpec((D, D), lambda i: (0, 0))],
        out_specs=[pl.BlockSpec((BM, D), lambda i: (i, 0)),
                   pl.BlockSpec((BM, D), lambda i: (i, 0))],
        out_shape=[jax.ShapeDtypeStruct((N, D), jnp.float32),
                   jax.ShapeDtypeStruct((N, D), jnp.float32)],
    )(x, w1, w2)


_MESH = plsc.VectorSubcoreMesh(core_axis_name="c", subcore_axis_name="s")


@functools.partial(
    pl.kernel,
    out_type=[
        jax.ShapeDtypeStruct((NC, NACC, D), jnp.float32),
        jax.ShapeDtypeStruct((NPAD, D), jnp.float32),
        jax.ShapeDtypeStruct((NPAD, D), jnp.float32),
    ],
    mesh=_MESH,
    scratch_types=[
        pltpu.VMEM((2, CHUNK), jnp.int32),
        pltpu.VMEM((CHUNK,), jnp.float32),
        pltpu.VMEM((CHUNK, D), jnp.float32),
        pltpu.VMEM((PCPT, PCH), jnp.int32),
        pltpu.VMEM_SHARED((NACC, D), jnp.float32),
        pltpu.SemaphoreType.DMA,
    ],
)
def _sc_spmm(h_hbm, re_hbm, g0_hbm, ecv_hbm, vals_hbm, perm_hbm,
             part_hbm, left_hbm, g0p_hbm,
             echunk, vchunk, gbuf, pidx_v, acc, sem):
    c = lax.axis_index("c")
    s = lax.axis_index("s")
    wid = s * NC + c

    def _zrow(i, carry):
        for q in range(D // L):
            gbuf[i, pl.ds(q * L, L)] = jnp.zeros((L,), jnp.float32)
        return carry
    lax.fori_loop(0, ZR, _zrow, 0)
    for k in range(RPS // ZR):
        pltpu.sync_copy(gbuf, acc.at[pl.ds(s * RPS + k * ZR, ZR)])

    pltpu.sync_copy(perm_hbm.at[wid], pidx_v)
    pgbuf = gbuf.at[pl.ds(0, PCH)]
    for j in range(PCPT):
        base = wid * PPT + j * PCH
        pltpu.async_copy(re_hbm.at[pidx_v.at[j]], pgbuf, sem).wait()
        pltpu.sync_copy(pgbuf, left_hbm.at[pl.ds(base, PCH)])
        pltpu.async_copy(g0_hbm.at[pidx_v.at[j]], pgbuf, sem).wait()
        pltpu.sync_copy(pgbuf, g0p_hbm.at[pl.ds(base, PCH)])

    plsc.subcore_barrier()

    def _chunk(j, carry):
        pltpu.sync_copy(ecv_hbm.at[wid, j], echunk)
        pltpu.sync_copy(vals_hbm.at[wid, j], vchunk)
        pltpu.async_copy(h_hbm.at[echunk.at[0]], gbuf, sem).wait()

        def _group(g, carry2):
            vv = vchunk[pl.ds(g * L, L)]
            for e in range(L):
                val = vv[e]
                r = g * L + e
                for q in range(D // L):
                    gbuf[r, pl.ds(q * L, L)] = gbuf[r, pl.ds(q * L, L)] * val
            return carry2
        lax.fori_loop(0, CHUNK // L, _group, 0)

        pltpu.sync_copy(gbuf, acc.at[echunk.at[1]], add=True)
        return carry
    lax.fori_loop(0, CPT, _chunk, 0)

    plsc.subcore_barrier()

    for k in range(RPS // ZR):
        r0 = s * RPS + k * ZR
        pltpu.sync_copy(acc.at[pl.ds(r0, ZR)], gbuf)
        pltpu.sync_copy(gbuf, part_hbm.at[c, pl.ds(r0, ZR)])


def _fuse_body(p_ref, left_ref, g0p_ref, b_ref, o_ref):
    gate = jax.nn.sigmoid(g0p_ref[...] + b_ref[...])
    agg = jnp.maximum(p_ref[0] + p_ref[1], 0.0)
    o_ref[...] = gate * agg + (1.0 - gate) * left_ref[...]


def _fuse(part, left, g0p, b):
    BM = 1000
    return pl.pallas_call(
        _fuse_body,
        grid=(N // BM,),
        in_specs=[pl.BlockSpec((NC, BM, D), lambda i: (0, i, 0)),
                  pl.BlockSpec((BM, D), lambda i: (i, 0)),
                  pl.BlockSpec((BM, D), lambda i: (i, 0)),
                  pl.BlockSpec((1, D), lambda i: (0, 0))],
        out_specs=pl.BlockSpec((BM, D), lambda i: (i, 0)),
        out_shape=jax.ShapeDtypeStruct((N, D), jnp.float32),
    )(part, left, g0p, b)


def kernel(right_embed, edge_index, adj_vals, perm, gcnW1, highwayWr, highwaybr):
    right_embed = right_embed.astype(jnp.float32)
    h, g0 = _mm2(right_embed, gcnW1.astype(jnp.float32),
                 highwayWr.astype(jnp.float32))

    epad = EPAD - E
    rows3 = jnp.pad(edge_index[0], (0, epad)).reshape(NW, CPT, CHUNK).astype(jnp.int32)
    cols3 = jnp.pad(edge_index[1], (0, epad)).reshape(NW, CPT, CHUNK).astype(jnp.int32)
    vals3 = jnp.pad(adj_vals.astype(jnp.float32), (0, epad)).reshape(NW, CPT, CHUNK)
    ecv = jnp.stack([cols3, rows3], axis=2)
    perm3 = jnp.pad(perm, (0, NPAD - N)).reshape(NW, PCPT, PCH).astype(jnp.int32)

    part, left_pad, g0p_pad = _sc_spmm(h, right_embed, g0, ecv, vals3, perm3)

    return _fuse(part[:, :N], left_pad[:N], g0p_pad[:N],
                 highwaybr.astype(jnp.float32).reshape(1, D))
